# baseline (speedup 1.0000x reference)
import functools

import jax
import jax.numpy as jnp
from jax import lax
from jax.experimental import pallas as pl
from jax.experimental.pallas import tpu as pltpu
from jax.experimental.pallas import tpu_sc as plsc

B = 16384
D = 64
H = 128

NC = 2
NS = 16
NW = NC * NS
BPW = B // NW


def _sc_gather(user, movie, utT, mtT):
  mesh = plsc.VectorSubcoreMesh(core_axis_name="c", subcore_axis_name="s")

  @functools.partial(
      pl.kernel,
      mesh=mesh,
      out_type=[
          jax.ShapeDtypeStruct((D, B), jnp.float32),
          jax.ShapeDtypeStruct((D, B), jnp.float32),
      ],
      scratch_types=[
          pltpu.VMEM((BPW,), jnp.int32),
          pltpu.VMEM((BPW,), jnp.int32),
          pltpu.VMEM((D, BPW), jnp.float32),
          pltpu.VMEM((D, BPW), jnp.float32),
          pltpu.SemaphoreType.DMA,
          pltpu.SemaphoreType.DMA,
      ],
      compiler_params=pltpu.CompilerParams(use_tc_tiling_on_sc=False),
  )
  def k(ur_hbm, mr_hbm, ut_hbm, mt_hbm, xu_hbm, xv_hbm,
        idx_u, idx_m, rows_u, rows_m, sem_u, sem_m):
    wid = lax.axis_index("s") * NC + lax.axis_index("c")
    base = wid * BPW
    pltpu.sync_copy(ur_hbm.at[pl.ds(base, BPW)], idx_u)
    pltpu.sync_copy(mr_hbm.at[pl.ds(base, BPW)], idx_m)
    waits = []
    for d in range(D):
      waits.append(pltpu.async_copy(
          ut_hbm.at[d].at[idx_u], rows_u.at[d], sem_u))
      waits.append(pltpu.async_copy(
          mt_hbm.at[d].at[idx_m], rows_m.at[d], sem_m))
    for cp in waits:
      cp.wait()
    pltpu.sync_copy(rows_u, xu_hbm.at[:, pl.ds(base, BPW)])
    pltpu.sync_copy(rows_m, xv_hbm.at[:, pl.ds(base, BPW)])

  return k(user, movie, utT, mtT)


BLK = 2048


def _mlp_body(xuT_ref, xvT_ref, w1aT_ref, w1bT_ref, b1c_ref, w2r_ref, b2_ref,
              out_ref):
  hT = jnp.dot(w1aT_ref[...], xuT_ref[...], preferred_element_type=jnp.float32)
  hT = hT + jnp.dot(w1bT_ref[...], xvT_ref[...],
                    preferred_element_type=jnp.float32)
  hT = jnp.maximum(hT + b1c_ref[...], 0.0)
  o = jnp.dot(w2r_ref[...], hT, preferred_element_type=jnp.float32)
  out_ref[...] = o[0] + b2_ref[0, 0]


def _mlp(xuT, xvT, w1aT, w1bT, b1c, w2r, b2r):
  return pl.pallas_call(
      _mlp_body,
      grid=(B // BLK,),
      in_specs=[
          pl.BlockSpec((D, BLK), lambda i: (0, i)),
          pl.BlockSpec((D, BLK), lambda i: (0, i)),
          pl.BlockSpec((H, D), lambda i: (0, 0)),
          pl.BlockSpec((H, D), lambda i: (0, 0)),
          pl.BlockSpec((H, 1), lambda i: (0, 0)),
          pl.BlockSpec((1, H), lambda i: (0, 0)),
          pl.BlockSpec((1, 1), lambda i: (0, 0)),
      ],
      out_specs=pl.BlockSpec((BLK,), lambda i: (i,)),
      out_shape=jax.ShapeDtypeStruct((B,), jnp.float32),
  )(xuT, xvT, w1aT, w1bT, b1c, w2r, b2r)


def kernel(user, movie, user_table, movie_table, W1, b1, W2, b2):
  xuT, xvT = _sc_gather(user, movie, user_table.T, movie_table.T)
  w1aT = W1[:D].T
  w1bT = W1[D:].T
  b1c = b1.reshape(H, 1)
  w2r = W2.reshape(1, H)
  b2r = b2.reshape(1, 1)
  return _mlp(xuT, xvT, w1aT, w1bT, b1c, w2r, b2r)


# speedup vs baseline: 8.8638x; 8.8638x over previous
import functools

import jax
import jax.numpy as jnp
from jax import lax
from jax.experimental import pallas as pl
from jax.experimental.pallas import tpu as pltpu
from jax.experimental.pallas import tpu_sc as plsc

B = 16384
D = 64
H = 128

NC = 2
NS = 16
NW = NC * NS
BPW = B // NW
HPW = BPW // 2


def _sc_gather(gu, gm, tu, tm):
  mesh = plsc.VectorSubcoreMesh(core_axis_name="c", subcore_axis_name="s")

  @functools.partial(
      pl.kernel,
      mesh=mesh,
      out_type=[
          jax.ShapeDtypeStruct((B, 2 * D), jnp.float32),
          jax.ShapeDtypeStruct((B, 2 * D), jnp.float32),
      ],
      scratch_types=[
          pltpu.VMEM((HPW,), jnp.int32),
          pltpu.VMEM((HPW,), jnp.int32),
          pltpu.VMEM((HPW,), jnp.int32),
          pltpu.VMEM((HPW,), jnp.int32),
          pltpu.VMEM((HPW, 2 * D), jnp.float32),
          pltpu.VMEM((HPW, 2 * D), jnp.float32),
          pltpu.SemaphoreType.DMA,
          pltpu.SemaphoreType.DMA,
      ],
      compiler_params=pltpu.CompilerParams(
          use_tc_tiling_on_sc=True, needs_layout_passes=True),
  )
  def k(gu_hbm, gm_hbm, tu_hbm, tm_hbm, xu_hbm, xv_hbm,
        iu0, iu1, im0, im1, rows_u, rows_m, sem_u, sem_m):
    wid = lax.axis_index("s") * NC + lax.axis_index("c")
    base = wid * BPW
    pltpu.sync_copy(gu_hbm.at[pl.ds(base, HPW)], iu0)
    pltpu.sync_copy(gu_hbm.at[pl.ds(base + HPW, HPW)], iu1)
    pltpu.sync_copy(gm_hbm.at[pl.ds(base, HPW)], im0)
    pltpu.sync_copy(gm_hbm.at[pl.ds(base + HPW, HPW)], im1)

    cu = pltpu.async_copy(tu_hbm.at[iu0], rows_u, sem_u)
    cm = pltpu.async_copy(tm_hbm.at[im0], rows_m, sem_m)
    cu.wait()
    pltpu.sync_copy(rows_u, xu_hbm.at[pl.ds(base, HPW)])
    cm.wait()
    pltpu.sync_copy(rows_m, xv_hbm.at[pl.ds(base, HPW)])

    cu = pltpu.async_copy(tu_hbm.at[iu1], rows_u, sem_u)
    cm = pltpu.async_copy(tm_hbm.at[im1], rows_m, sem_m)
    cu.wait()
    pltpu.sync_copy(rows_u, xu_hbm.at[pl.ds(base + HPW, HPW)])
    cm.wait()
    pltpu.sync_copy(rows_m, xv_hbm.at[pl.ds(base + HPW, HPW)])

  return k(gu, gm, tu, tm)


BLK = 2048


def _mlp_body(xu_ref, xv_ref, pu_ref, pv_ref, w1a_ref, w1b_ref, b1_ref,
              w2_ref, b2_ref, out_ref):
  xu = xu_ref[...]
  xv = xv_ref[...]
  pu = pu_ref[...]
  pv = pv_ref[...]
  su = xu[:, :D] + pu * (xu[:, D:] - xu[:, :D])
  sv = xv[:, :D] + pv * (xv[:, D:] - xv[:, :D])
  h = jnp.dot(su, w1a_ref[...], preferred_element_type=jnp.float32)
  h = h + jnp.dot(sv, w1b_ref[...], preferred_element_type=jnp.float32)
  h = jnp.maximum(h + b1_ref[...], 0.0)
  o = jnp.dot(h, w2_ref[...], preferred_element_type=jnp.float32)
  out_ref[...] = o[:, 0] + b2_ref[0, 0]


def _mlp(xu, xv, pu, pv, w1a, w1b, b1r, w2, b2r):
  return pl.pallas_call(
      _mlp_body,
      grid=(B // BLK,),
      in_specs=[
          pl.BlockSpec((BLK, 2 * D), lambda i: (i, 0)),
          pl.BlockSpec((BLK, 2 * D), lambda i: (i, 0)),
          pl.BlockSpec((BLK, 1), lambda i: (i, 0)),
          pl.BlockSpec((BLK, 1), lambda i: (i, 0)),
          pl.BlockSpec((D, H), lambda i: (0, 0)),
          pl.BlockSpec((D, H), lambda i: (0, 0)),
          pl.BlockSpec((1, H), lambda i: (0, 0)),
          pl.BlockSpec((H, 1), lambda i: (0, 0)),
          pl.BlockSpec((1, 1), lambda i: (0, 0)),
      ],
      out_specs=pl.BlockSpec((BLK,), lambda i: (i,)),
      out_shape=jax.ShapeDtypeStruct((B,), jnp.float32),
  )(xu, xv, pu, pv, w1a, w1b, b1r, w2, b2r)


def kernel(user, movie, user_table, movie_table, W1, b1, W2, b2):
  gu = jax.lax.shift_right_logical(user, 1)
  gm = jax.lax.shift_right_logical(movie, 1)
  pu = jnp.bitwise_and(user, 1).astype(jnp.float32).reshape(B, 1)
  pv = jnp.bitwise_and(movie, 1).astype(jnp.float32).reshape(B, 1)
  tu = user_table.reshape(B * 0 + 500000, 2 * D)
  tm = movie_table.reshape(500000, 2 * D)
  xu, xv = _sc_gather(gu, gm, tu, tm)
  w1a = W1[:D]
  w1b = W1[D:]
  b1r = b1.reshape(1, H)
  b2r = b2.reshape(1, 1)
  return _mlp(xu, xv, pu, pv, w1a, w1b, b1r, W2, b2r)


# speedup vs baseline: 8.8660x; 1.0003x over previous
import functools

import jax
import jax.numpy as jnp
from jax import lax
from jax.experimental import pallas as pl
from jax.experimental.pallas import tpu as pltpu
from jax.experimental.pallas import tpu_sc as plsc

B = 16384
D = 64
H = 128

NC = 2
NS = 16
NW = NC * NS
BPW = B // NW
HPW = BPW // 2


def _sc_gather(gu, gm, tu, tm):
  mesh = plsc.VectorSubcoreMesh(core_axis_name="c", subcore_axis_name="s")

  @functools.partial(
      pl.kernel,
      mesh=mesh,
      out_type=[
          jax.ShapeDtypeStruct((B, 2 * D), jnp.float32),
          jax.ShapeDtypeStruct((B, 2 * D), jnp.float32),
      ],
      scratch_types=[
          pltpu.VMEM((HPW,), jnp.int32),
          pltpu.VMEM((HPW,), jnp.int32),
          pltpu.VMEM((HPW,), jnp.int32),
          pltpu.VMEM((HPW,), jnp.int32),
          pltpu.VMEM((HPW, 2 * D), jnp.float32),
          pltpu.VMEM((HPW, 2 * D), jnp.float32),
          pltpu.SemaphoreType.DMA,
          pltpu.SemaphoreType.DMA,
      ],
      compiler_params=pltpu.CompilerParams(
          use_tc_tiling_on_sc=False, needs_layout_passes=True),
  )
  def k(gu_hbm, gm_hbm, tu_hbm, tm_hbm, xu_hbm, xv_hbm,
        iu0, iu1, im0, im1, rows_u, rows_m, sem_u, sem_m):
    wid = lax.axis_index("s") * NC + lax.axis_index("c")
    base = wid * BPW
    pltpu.sync_copy(gu_hbm.at[pl.ds(base, HPW)], iu0)
    pltpu.sync_copy(gu_hbm.at[pl.ds(base + HPW, HPW)], iu1)
    pltpu.sync_copy(gm_hbm.at[pl.ds(base, HPW)], im0)
    pltpu.sync_copy(gm_hbm.at[pl.ds(base + HPW, HPW)], im1)

    cu = pltpu.async_copy(tu_hbm.at[iu0], rows_u, sem_u)
    cm = pltpu.async_copy(tm_hbm.at[im0], rows_m, sem_m)
    cu.wait()
    pltpu.sync_copy(rows_u, xu_hbm.at[pl.ds(base, HPW)])
    cm.wait()
    pltpu.sync_copy(rows_m, xv_hbm.at[pl.ds(base, HPW)])

    cu = pltpu.async_copy(tu_hbm.at[iu1], rows_u, sem_u)
    cm = pltpu.async_copy(tm_hbm.at[im1], rows_m, sem_m)
    cu.wait()
    pltpu.sync_copy(rows_u, xu_hbm.at[pl.ds(base + HPW, HPW)])
    cm.wait()
    pltpu.sync_copy(rows_m, xv_hbm.at[pl.ds(base + HPW, HPW)])

  return k(gu, gm, tu, tm)


BLK = 2048


def _mlp_body(xu_ref, xv_ref, pu_ref, pv_ref, w1a_ref, w1b_ref, b1_ref,
              w2_ref, b2_ref, out_ref):
  xu = xu_ref[...]
  xv = xv_ref[...]
  pu = pu_ref[...]
  pv = pv_ref[...]
  su = xu[:, :D] + pu * (xu[:, D:] - xu[:, :D])
  sv = xv[:, :D] + pv * (xv[:, D:] - xv[:, :D])
  h = jnp.dot(su, w1a_ref[...], preferred_element_type=jnp.float32)
  h = h + jnp.dot(sv, w1b_ref[...], preferred_element_type=jnp.float32)
  h = jnp.maximum(h + b1_ref[...], 0.0)
  o = jnp.dot(h, w2_ref[...], preferred_element_type=jnp.float32)
  out_ref[...] = o[:, 0] + b2_ref[0, 0]


def _mlp(xu, xv, pu, pv, w1a, w1b, b1r, w2, b2r):
  return pl.pallas_call(
      _mlp_body,
      grid=(B // BLK,),
      in_specs=[
          pl.BlockSpec((BLK, 2 * D), lambda i: (i, 0)),
          pl.BlockSpec((BLK, 2 * D), lambda i: (i, 0)),
          pl.BlockSpec((BLK, 1), lambda i: (i, 0)),
          pl.BlockSpec((BLK, 1), lambda i: (i, 0)),
          pl.BlockSpec((D, H), lambda i: (0, 0)),
          pl.BlockSpec((D, H), lambda i: (0, 0)),
          pl.BlockSpec((1, H), lambda i: (0, 0)),
          pl.BlockSpec((H, 1), lambda i: (0, 0)),
          pl.BlockSpec((1, 1), lambda i: (0, 0)),
      ],
      out_specs=pl.BlockSpec((BLK,), lambda i: (i,)),
      out_shape=jax.ShapeDtypeStruct((B,), jnp.float32),
  )(xu, xv, pu, pv, w1a, w1b, b1r, w2, b2r)


def kernel(user, movie, user_table, movie_table, W1, b1, W2, b2):
  gu = jax.lax.shift_right_logical(user, 1)
  gm = jax.lax.shift_right_logical(movie, 1)
  pu = jnp.bitwise_and(user, 1).astype(jnp.float32).reshape(B, 1)
  pv = jnp.bitwise_and(movie, 1).astype(jnp.float32).reshape(B, 1)
  tu = user_table.reshape(B * 0 + 500000, 2 * D)
  tm = movie_table.reshape(500000, 2 * D)
  xu, xv = _sc_gather(gu, gm, tu, tm)
  w1a = W1[:D]
  w1b = W1[D:]
  b1r = b1.reshape(1, H)
  b2r = b2.reshape(1, 1)
  return _mlp(xu, xv, pu, pv, w1a, w1b, b1r, W2, b2r)


# speedup vs baseline: 8.9190x; 1.0060x over previous
import functools

import jax
import jax.numpy as jnp
from jax import lax
from jax.experimental import pallas as pl
from jax.experimental.pallas import tpu as pltpu
from jax.experimental.pallas import tpu_sc as plsc

B = 16384
D = 64
H = 128

NC = 2
NS = 16
NW = NC * NS
BPW = B // NW


def _sc_gather(user, movie, ut, mt):
  mesh = plsc.VectorSubcoreMesh(core_axis_name="c", subcore_axis_name="s")

  @functools.partial(
      pl.kernel,
      mesh=mesh,
      out_type=[
          jax.ShapeDtypeStruct((B, D), jnp.float32),
          jax.ShapeDtypeStruct((B, D), jnp.float32),
      ],
      scratch_types=[
          pltpu.VMEM((BPW,), jnp.int32),
          pltpu.VMEM((BPW,), jnp.int32),
          pltpu.VMEM((BPW, D), jnp.float32),
          pltpu.VMEM((BPW, D), jnp.float32),
          pltpu.SemaphoreType.DMA,
          pltpu.SemaphoreType.DMA,
      ],
      compiler_params=pltpu.CompilerParams(use_tc_tiling_on_sc=False),
  )
  def k(ur_hbm, mr_hbm, ut_hbm, mt_hbm, xu_hbm, xv_hbm,
        idx_u, idx_m, rows_u, rows_m, sem_u, sem_m):
    wid = lax.axis_index("s") * NC + lax.axis_index("c")
    base = wid * BPW
    pltpu.sync_copy(ur_hbm.at[pl.ds(base, BPW)], idx_u)
    pltpu.sync_copy(mr_hbm.at[pl.ds(base, BPW)], idx_m)
    cu = pltpu.async_copy(ut_hbm.at[idx_u], rows_u, sem_u)
    cm = pltpu.async_copy(mt_hbm.at[idx_m], rows_m, sem_m)
    cu.wait()
    cm.wait()
    pltpu.sync_copy(rows_u, xu_hbm.at[pl.ds(base, BPW)])
    pltpu.sync_copy(rows_m, xv_hbm.at[pl.ds(base, BPW)])

  return k(user, movie, ut, mt)


BLK = 2048


def _mlp_body(xu_ref, xv_ref, w1a_ref, w1b_ref, b1_ref, w2_ref, b2_ref,
              out_ref):
  h = jnp.dot(xu_ref[...], w1a_ref[...], preferred_element_type=jnp.float32)
  h = h + jnp.dot(xv_ref[...], w1b_ref[...],
                  preferred_element_type=jnp.float32)
  h = jnp.maximum(h + b1_ref[...], 0.0)
  o = jnp.dot(h, w2_ref[...], preferred_element_type=jnp.float32)
  out_ref[...] = o[:, 0] + b2_ref[0, 0]


def _mlp(xu, xv, w1a, w1b, b1r, w2, b2r):
  return pl.pallas_call(
      _mlp_body,
      grid=(B // BLK,),
      in_specs=[
          pl.BlockSpec((BLK, D), lambda i: (i, 0)),
          pl.BlockSpec((BLK, D), lambda i: (i, 0)),
          pl.BlockSpec((D, H), lambda i: (0, 0)),
          pl.BlockSpec((D, H), lambda i: (0, 0)),
          pl.BlockSpec((1, H), lambda i: (0, 0)),
          pl.BlockSpec((H, 1), lambda i: (0, 0)),
          pl.BlockSpec((1, 1), lambda i: (0, 0)),
      ],
      out_specs=pl.BlockSpec((BLK,), lambda i: (i,)),
      out_shape=jax.ShapeDtypeStruct((B,), jnp.float32),
  )(xu, xv, w1a, w1b, b1r, w2, b2r)


def kernel(user, movie, user_table, movie_table, W1, b1, W2, b2):
  xu, xv = _sc_gather(user, movie, user_table, movie_table)
  w1a = W1[:D]
  w1b = W1[D:]
  b1r = b1.reshape(1, H)
  b2r = b2.reshape(1, 1)
  return _mlp(xu, xv, w1a, w1b, b1r, W2, b2r)
